# baseline (speedup 1.0000x reference)
import functools

import jax
import jax.numpy as jnp
from jax import lax
from jax.experimental import pallas as pl
from jax.experimental.pallas import tpu as pltpu
from jax.experimental.pallas import tpu_sc as plsc

MARGIN = 0.5
NON_MATCH_WEIGHT = 1.0

NC = 2
NS = 16
NW = NC * NS
L = 16
CHUNK = 128


def _ceil_to(x, m):
    return ((x + m - 1) // m) * m


def _make_kernel(n_rows, d, batch, nb_match, nb_nonmatch, ppw_m, ppw_nm):
    n = n_rows // batch
    chunks_m = ppw_m // CHUNK
    chunks_nm = ppw_nm // CHUNK
    total_m = batch * nb_match
    total_nm = batch * nb_nonmatch

    mesh = plsc.VectorSubcoreMesh(core_axis_name="c", subcore_axis_name="s")

    @functools.partial(
        pl.kernel,
        out_type=jax.ShapeDtypeStruct((NC, L), jnp.float32),
        mesh=mesh,
        scratch_types=[
            pltpu.VMEM((ppw_nm,), jnp.int32),
            pltpu.VMEM((ppw_nm,), jnp.int32),
            pltpu.VMEM((CHUNK, L), jnp.float32),
            pltpu.VMEM((CHUNK, L), jnp.float32),
            pltpu.VMEM((2, L), jnp.float32),
            pltpu.VMEM_SHARED((NS, 2, L), jnp.float32),
            pltpu.VMEM((NS, 2, L), jnp.float32),
            pltpu.VMEM((L,), jnp.float32),
            pltpu.SemaphoreType.DMA,
        ],
        compiler_params=pltpu.CompilerParams(use_tc_tiling_on_sc=False,
                                             needs_layout_passes=False),
    )
    def launch(a_hbm, b_hbm, ma_hbm, mb_hbm, na_hbm, nb_hbm, out_hbm,
               idxa_v, idxb_v, rows_a, rows_b, acc_v, shared, red_v, ob_v,
               sem):
        cid = lax.axis_index("c")
        sid = lax.axis_index("s")
        wid = cid * NS + sid
        il = lax.iota(jnp.int32, L)
        nfull = jnp.full((L,), n, jnp.int32)
        izero = jnp.zeros((L,), jnp.int32)
        fzero = jnp.zeros((L,), jnp.float32)

        def run_phase(ia_hbm, ib_hbm, ppw, n_chunks, nb, total_real, is_match):
            base = wid * ppw
            pltpu.sync_copy(ia_hbm.at[pl.ds(base, ppw)],
                            idxa_v.at[pl.ds(0, ppw)])
            pltpu.sync_copy(ib_hbm.at[pl.ds(base, ppw)],
                            idxb_v.at[pl.ds(0, ppw)])

            def chunk_body(c, acc):
                for g in range(CHUNK // L):
                    off = c * CHUNK + g * L
                    jv = base + off + il
                    boff = izero
                    for k in range(1, batch):
                        boff = boff + jnp.where(jv >= k * nb, nfull, izero)
                    idxa_v[pl.ds(off, L)] = idxa_v[pl.ds(off, L)] + boff
                    idxb_v[pl.ds(off, L)] = idxb_v[pl.ds(off, L)] + boff
                cpy_a = pltpu.async_copy(
                    a_hbm.at[idxa_v.at[pl.ds(c * CHUNK, CHUNK)]], rows_a, sem)
                cpy_b = pltpu.async_copy(
                    b_hbm.at[idxb_v.at[pl.ds(c * CHUNK, CHUNK)]], rows_b, sem)
                cpy_a.wait()
                cpy_b.wait()
                for g in range(CHUNK // L):
                    pvec = g * L + il
                    dist = fzero
                    for dd in range(d):
                        dvec = jnp.full((L,), dd, jnp.int32)
                        va = plsc.load_gather(rows_a, [pvec, dvec])
                        vb = plsc.load_gather(rows_b, [pvec, dvec])
                        t = va - vb
                        dist = dist + t * t
                    jv = base + c * CHUNK + g * L + il
                    valid = jv < total_real
                    if is_match:
                        contrib = dist
                    else:
                        contrib = jnp.maximum(MARGIN - dist, 0.0)
                    acc = acc + jnp.where(valid, contrib, fzero)
                return acc

            return lax.fori_loop(0, n_chunks, chunk_body, fzero)

        macc = run_phase(ma_hbm, mb_hbm, ppw_m, chunks_m, nb_match,
                         total_m, True)
        nmacc = run_phase(na_hbm, nb_hbm, ppw_nm, chunks_nm, nb_nonmatch,
                          total_nm, False)

        acc_v[0] = macc * jnp.float32(1.0 / nb_match)
        acc_v[1] = nmacc * jnp.float32(NON_MATCH_WEIGHT / nb_nonmatch)
        pltpu.sync_copy(acc_v, shared.at[sid])
        plsc.subcore_barrier()

        @pl.when(sid == 0)
        def _():
            pltpu.sync_copy(shared, red_v)
            m_tot = fzero
            nm_tot = fzero
            for i in range(NS):
                m_tot = m_tot + red_v[i, 0]
                nm_tot = nm_tot + red_v[i, 1]
            m_s = jnp.sum(m_tot)
            nm_s = jnp.sum(nm_tot)
            ovec = jnp.where(il == 0, lax.broadcast(m_s, (L,)),
                             jnp.where(il == 1, lax.broadcast(nm_s, (L,)),
                                       fzero))
            ob_v[...] = ovec
            pltpu.sync_copy(ob_v, out_hbm.at[cid])

    return launch


def kernel(outA, outB, matchA, matchB, nonMatchA, nonMatchB, device):
    b, _, n, d = outA.shape
    nb_match = matchA.shape[1]
    nb_nonmatch = nonMatchA.shape[1]

    a2d = outA.reshape(b * n, d)
    b2d = outB.reshape(b * n, d)

    ppw_m = _ceil_to(_ceil_to(b * nb_match, NW) // NW, 2 * CHUNK)
    ppw_nm = _ceil_to(_ceil_to(b * nb_nonmatch, NW) // NW, 2 * CHUNK)

    def flat_pad(x, ppw):
        f = x.reshape(-1).astype(jnp.int32)
        return jnp.pad(f, (0, NW * ppw - f.shape[0]))

    ma = flat_pad(matchA, ppw_m)
    mb = flat_pad(matchB, ppw_m)
    na = flat_pad(nonMatchA, ppw_nm)
    nb = flat_pad(nonMatchB, ppw_nm)

    launch = _make_kernel(b * n, d, b, nb_match, nb_nonmatch, ppw_m, ppw_nm)
    out = launch(a2d, b2d, ma, mb, na, nb)

    m_loss = out[0, 0] + out[1, 0]
    nm_loss = out[0, 1] + out[1, 1]
    return (m_loss + nm_loss, m_loss, nm_loss)


# speedup vs baseline: 1.0570x; 1.0570x over previous
import functools

import jax
import jax.numpy as jnp
from jax import lax
from jax.experimental import pallas as pl
from jax.experimental.pallas import tpu as pltpu
from jax.experimental.pallas import tpu_sc as plsc

MARGIN = 0.5
NON_MATCH_WEIGHT = 1.0

NC = 2
NS = 16
NW = NC * NS
L = 16
CHUNK = 128
SEG_NM = 5
NSEG_NM = 10
SEG_M = 3
NSEG_M = 2
PPW_NM = SEG_NM * NSEG_NM * CHUNK
PPW_M = SEG_M * NSEG_M * CHUNK
RBUF = max(SEG_NM, SEG_M) * CHUNK


def _make_kernel(n_rows, d, batch, nb_match, nb_nonmatch):
    n = n_rows // batch
    total_m = batch * nb_match
    total_nm = batch * nb_nonmatch

    mesh = plsc.VectorSubcoreMesh(core_axis_name="c", subcore_axis_name="s")

    @functools.partial(
        pl.kernel,
        out_type=jax.ShapeDtypeStruct((NC, L), jnp.float32),
        mesh=mesh,
        scratch_types=[
            pltpu.VMEM((PPW_NM,), jnp.int32),
            pltpu.VMEM((PPW_NM,), jnp.int32),
            pltpu.VMEM((RBUF, L), jnp.float32),
            pltpu.VMEM((RBUF, L), jnp.float32),
            pltpu.VMEM((RBUF, L), jnp.float32),
            pltpu.VMEM((RBUF, L), jnp.float32),
            pltpu.VMEM((2, L), jnp.float32),
            pltpu.VMEM_SHARED((NS, 2, L), jnp.float32),
            pltpu.VMEM((NS, 2, L), jnp.float32),
            pltpu.VMEM((L,), jnp.float32),
            pltpu.SemaphoreType.DMA,
            pltpu.SemaphoreType.DMA,
        ],
        compiler_params=pltpu.CompilerParams(use_tc_tiling_on_sc=False,
                                             needs_layout_passes=False),
    )
    def launch(a_hbm, b_hbm, ma_hbm, mb_hbm, na_hbm, nb_hbm, out_hbm,
               idxa_v, idxb_v, ra0, rb0, ra1, rb1, acc_v, shared, red_v,
               ob_v, sem0, sem1):
        cid = lax.axis_index("c")
        sid = lax.axis_index("s")
        wid = cid * NS + sid
        il = lax.iota(jnp.int32, L)
        nfull = jnp.full((L,), n, jnp.int32)
        izero = jnp.zeros((L,), jnp.int32)
        fzero = jnp.zeros((L,), jnp.float32)

        def run_phase(ia_hbm, ib_hbm, ppw, seg_ch, nseg, nb, total_real,
                      is_match):
            base = wid * ppw
            pltpu.sync_copy(ia_hbm.at[pl.ds(base, ppw)],
                            idxa_v.at[pl.ds(0, ppw)])
            pltpu.sync_copy(ib_hbm.at[pl.ds(base, ppw)],
                            idxb_v.at[pl.ds(0, ppw)])

            def adjust(s, carry):
                off = s * L
                jv = base + off + il
                boff = izero
                for k in range(1, batch):
                    boff = boff + jnp.where(jv >= k * nb, nfull, izero)
                idxa_v[pl.ds(off, L)] = idxa_v[pl.ds(off, L)] + boff
                idxb_v[pl.ds(off, L)] = idxb_v[pl.ds(off, L)] + boff
                return carry

            lax.fori_loop(0, ppw // L, adjust, 0)

            def fire(s, bufa, bufb, sem):
                c0 = s * seg_ch
                for k in range(seg_ch):
                    pltpu.async_copy(
                        a_hbm.at[idxa_v.at[pl.ds((c0 + k) * CHUNK, CHUNK)]],
                        bufa.at[pl.ds(k * CHUNK, CHUNK)], sem)
                    pltpu.async_copy(
                        b_hbm.at[idxb_v.at[pl.ds((c0 + k) * CHUNK, CHUNK)]],
                        bufb.at[pl.ds(k * CHUNK, CHUNK)], sem)

            def drain(bufa, bufb, sem):
                for k in range(seg_ch):
                    pltpu.make_async_copy(
                        a_hbm.at[idxa_v.at[pl.ds(k * CHUNK, CHUNK)]],
                        bufa.at[pl.ds(k * CHUNK, CHUNK)], sem).wait()
                    pltpu.make_async_copy(
                        b_hbm.at[idxb_v.at[pl.ds(k * CHUNK, CHUNK)]],
                        bufb.at[pl.ds(k * CHUNK, CHUNK)], sem).wait()

            def compute_seg(s, bufa, bufb, acc):
                c0 = s * seg_ch

                def chunk_fn(k, acc):
                    for g in range(CHUNK // L):
                        pvec = k * CHUNK + g * L + il
                        dist = fzero
                        for dd in range(d):
                            dvec = jnp.full((L,), dd, jnp.int32)
                            va = plsc.load_gather(bufa, [pvec, dvec])
                            vb = plsc.load_gather(bufb, [pvec, dvec])
                            t = va - vb
                            dist = dist + t * t
                        jv = base + (c0 + k) * CHUNK + g * L + il
                        valid = jv < total_real
                        if is_match:
                            contrib = dist
                        else:
                            contrib = jnp.maximum(MARGIN - dist, 0.0)
                        acc = acc + jnp.where(valid, contrib, fzero)
                    return acc

                return lax.fori_loop(0, seg_ch, chunk_fn, acc)

            fire(0, ra0, rb0, sem0)

            def body(i, acc):
                fire(2 * i + 1, ra1, rb1, sem1)
                drain(ra0, rb0, sem0)
                acc = compute_seg(2 * i, ra0, rb0, acc)

                @pl.when(2 * i + 2 < nseg)
                def _():
                    fire(2 * i + 2, ra0, rb0, sem0)

                drain(ra1, rb1, sem1)
                acc = compute_seg(2 * i + 1, ra1, rb1, acc)
                return acc

            return lax.fori_loop(0, nseg // 2, body, fzero)

        macc = run_phase(ma_hbm, mb_hbm, PPW_M, SEG_M, NSEG_M, nb_match,
                         total_m, True)
        nmacc = run_phase(na_hbm, nb_hbm, PPW_NM, SEG_NM, NSEG_NM,
                          nb_nonmatch, total_nm, False)

        acc_v[0] = macc * jnp.float32(1.0 / nb_match)
        acc_v[1] = nmacc * jnp.float32(NON_MATCH_WEIGHT / nb_nonmatch)
        pltpu.sync_copy(acc_v, shared.at[sid])
        plsc.subcore_barrier()

        @pl.when(sid == 0)
        def _():
            pltpu.sync_copy(shared, red_v)
            m_tot = fzero
            nm_tot = fzero
            for i in range(NS):
                m_tot = m_tot + red_v[i, 0]
                nm_tot = nm_tot + red_v[i, 1]
            m_s = jnp.sum(m_tot)
            nm_s = jnp.sum(nm_tot)
            ovec = jnp.where(il == 0, lax.broadcast(m_s, (L,)),
                             jnp.where(il == 1, lax.broadcast(nm_s, (L,)),
                                       fzero))
            ob_v[...] = ovec
            pltpu.sync_copy(ob_v, out_hbm.at[cid])

    return launch


def kernel(outA, outB, matchA, matchB, nonMatchA, nonMatchB, device):
    b, _, n, d = outA.shape
    nb_match = matchA.shape[1]
    nb_nonmatch = nonMatchA.shape[1]

    a2d = outA.reshape(b * n, d)
    b2d = outB.reshape(b * n, d)

    def flat_pad(x, ppw):
        f = x.reshape(-1).astype(jnp.int32)
        return jnp.pad(f, (0, NW * ppw - f.shape[0]))

    ma = flat_pad(matchA, PPW_M)
    mb = flat_pad(matchB, PPW_M)
    na = flat_pad(nonMatchA, PPW_NM)
    nb = flat_pad(nonMatchB, PPW_NM)

    launch = _make_kernel(b * n, d, b, nb_match, nb_nonmatch)
    out = launch(a2d, b2d, ma, mb, na, nb)

    m_loss = out[0, 0] + out[1, 0]
    nm_loss = out[0, 1] + out[1, 1]
    return (m_loss + nm_loss, m_loss, nm_loss)
